# baseline (speedup 1.0000x reference)
import functools

import jax
import jax.numpy as jnp
from jax import lax
from jax.experimental import pallas as pl
from jax.experimental.pallas import tpu as pltpu
from jax.experimental.pallas import tpu_sc as plsc

N_ROWS = 16384
N_COLS = 2048
NC = 2
NS = 16
NW = NC * NS
ROWS_PER_W = N_ROWS // NW
R = 8
NB = ROWS_PER_W // R
JV = N_COLS // 16


def _body(z_hbm, perm_hbm, out_hbm, perm_v, in_v, out_v):
    wid = lax.axis_index("s") * NC + lax.axis_index("c")
    base = wid * ROWS_PER_W

    pltpu.sync_copy(perm_hbm, perm_v)

    row_ids = [jnp.full((16,), r, jnp.int32) for r in range(R)]

    def block(b, _):
        row0 = base + b * R
        pltpu.sync_copy(z_hbm.at[pl.ds(row0, R)], in_v)

        def jvec(jv, _):
            col0 = jv * 16
            idx = perm_v[pl.ds(col0, 16)]
            for r in range(R):
                vals = plsc.load_gather(in_v, [row_ids[r], idx])
                out_v[r, pl.ds(col0, 16)] = vals
            return 0

        lax.fori_loop(0, JV, jvec, 0)
        pltpu.sync_copy(out_v, out_hbm.at[pl.ds(row0, R)])
        return 0

    lax.fori_loop(0, NB, block, 0)


@jax.jit
def _permute(z, perm_i32):
    k = functools.partial(
        pl.kernel,
        out_type=jax.ShapeDtypeStruct((N_ROWS, N_COLS), jnp.float32),
        mesh=plsc.VectorSubcoreMesh(core_axis_name="c", subcore_axis_name="s"),
        scratch_types=[
            pltpu.VMEM((N_COLS,), jnp.int32),
            pltpu.VMEM((R, N_COLS), jnp.float32),
            pltpu.VMEM((R, N_COLS), jnp.float32),
        ],
        compiler_params=pltpu.CompilerParams(
            use_tc_tiling_on_sc=False, needs_layout_passes=False
        ),
    )(_body)
    return k(z, perm_i32)


def kernel(z, perm):
    return _permute(z, perm.astype(jnp.int32))


# speedup vs baseline: 1.2440x; 1.2440x over previous
import functools

import jax
import jax.numpy as jnp
from jax import lax
from jax.experimental import pallas as pl
from jax.experimental.pallas import tpu as pltpu
from jax.experimental.pallas import tpu_sc as plsc

N_ROWS = 16384
N_COLS = 2048
NC = 2
NS = 16
NW = NC * NS
ROWS_PER_W = N_ROWS // NW
R = 8
NB = ROWS_PER_W // R
JV = N_COLS // 16


def _body(z_hbm, perm_hbm, out_hbm,
          perm_v, in_v0, in_v1, out_v0, out_v1,
          sin0, sin1, sout0, sout1):
    wid = lax.axis_index("s") * NC + lax.axis_index("c")
    base = wid * ROWS_PER_W

    pltpu.sync_copy(perm_hbm, perm_v)

    in_bufs = (in_v0, in_v1)
    out_bufs = (out_v0, out_v1)
    sins = (sin0, sin1)
    souts = (sout0, sout1)

    row_ids = [jnp.full((16,), r, jnp.int32) for r in range(R)]

    def start_in(b, d):
        pltpu.async_copy(z_hbm.at[pl.ds(base + b * R, R)], in_bufs[d], sins[d])

    def wait_in(b, d):
        pltpu.make_async_copy(
            z_hbm.at[pl.ds(base + b * R, R)], in_bufs[d], sins[d]).wait()

    def start_out(b, d):
        pltpu.async_copy(out_bufs[d], out_hbm.at[pl.ds(base + b * R, R)],
                         souts[d])

    def wait_out(b, d):
        pltpu.make_async_copy(
            out_bufs[d], out_hbm.at[pl.ds(base + b * R, R)], souts[d]).wait()

    start_in(0, 0)

    def block_pair(b2, _):
        for d in range(2):
            b = 2 * b2 + d
            wait_in(b, d)

            @pl.when(b + 1 < NB)
            def _():
                start_in(b + 1, 1 - d)

            @pl.when(b2 > 0)
            def _():
                wait_out(b - 2, d)

            def jvec(jv, _):
                col0 = jv * 16
                idx = perm_v[pl.ds(col0, 16)]
                ov = out_bufs[d]
                iv = in_bufs[d]
                for r in range(R):
                    ov[r, pl.ds(col0, 16)] = plsc.load_gather(
                        iv, [row_ids[r], idx])
                return 0

            lax.fori_loop(0, JV, jvec, 0)
            start_out(b, d)
        return 0

    lax.fori_loop(0, NB // 2, block_pair, 0)
    wait_out(NB - 2, 0)
    wait_out(NB - 1, 1)


@jax.jit
def _permute(z, perm_i32):
    k = functools.partial(
        pl.kernel,
        out_type=jax.ShapeDtypeStruct((N_ROWS, N_COLS), jnp.float32),
        mesh=plsc.VectorSubcoreMesh(core_axis_name="c", subcore_axis_name="s"),
        scratch_types=[
            pltpu.VMEM((N_COLS,), jnp.int32),
            pltpu.VMEM((R, N_COLS), jnp.float32),
            pltpu.VMEM((R, N_COLS), jnp.float32),
            pltpu.VMEM((R, N_COLS), jnp.float32),
            pltpu.VMEM((R, N_COLS), jnp.float32),
            pltpu.SemaphoreType.DMA,
            pltpu.SemaphoreType.DMA,
            pltpu.SemaphoreType.DMA,
            pltpu.SemaphoreType.DMA,
        ],
        compiler_params=pltpu.CompilerParams(
            use_tc_tiling_on_sc=False, needs_layout_passes=False
        ),
    )(_body)
    return k(z, perm_i32)


def kernel(z, perm):
    return _permute(z, perm.astype(jnp.int32))


# speedup vs baseline: 1.8610x; 1.4960x over previous
import functools

import jax
import jax.numpy as jnp
from jax import lax
from jax.experimental import pallas as pl
from jax.experimental.pallas import tpu as pltpu
from jax.experimental.pallas import tpu_sc as plsc

N_ROWS = 16384
N_COLS = 2048
NC = 2
NS = 16
NW = NC * NS
ROWS_PER_W = N_ROWS // NW
R = 8
NB = ROWS_PER_W // R
JV = N_COLS // 16


def _body(z_hbm, perm_hbm, out_hbm,
          perm_v, in_v0, in_v1, out_v0, out_v1,
          sin0, sin1, sout0, sout1):
    wid = lax.axis_index("s") * NC + lax.axis_index("c")
    base = wid * ROWS_PER_W

    pltpu.sync_copy(perm_hbm, perm_v)

    in_bufs = (in_v0, in_v1)
    out_bufs = (out_v0, out_v1)
    sins = (sin0, sin1)
    souts = (sout0, sout1)

    row_ids = [jnp.full((16,), r, jnp.int32) for r in range(R)]

    def start_in(b, d):
        pltpu.async_copy(z_hbm.at[pl.ds(base + b * R, R)], in_bufs[d], sins[d])

    def wait_in(b, d):
        pltpu.make_async_copy(
            z_hbm.at[pl.ds(base + b * R, R)], in_bufs[d], sins[d]).wait()

    def start_out(b, d):
        pltpu.async_copy(out_bufs[d], out_hbm.at[pl.ds(base + b * R, R)],
                         souts[d])

    def wait_out(b, d):
        pltpu.make_async_copy(
            out_bufs[d], out_hbm.at[pl.ds(base + b * R, R)], souts[d]).wait()

    start_in(0, 0)

    def block_pair(b2, _):
        for d in range(2):
            b = 2 * b2 + d
            wait_in(b, d)

            @pl.when(b + 1 < NB)
            def _():
                start_in(b + 1, 1 - d)

            @pl.when(b2 > 0)
            def _():
                wait_out(b - 2, d)

            ov = out_bufs[d]
            iv = in_bufs[d]

            @plsc.parallel_loop(0, JV, 1, unroll=4)
            def jvec(jv):
                col0 = jv * 16
                idx = perm_v[pl.ds(col0, 16)]
                for r in range(R):
                    ov[r, pl.ds(col0, 16)] = plsc.load_gather(
                        iv, [row_ids[r], idx])
            start_out(b, d)
        return 0

    lax.fori_loop(0, NB // 2, block_pair, 0)
    wait_out(NB - 2, 0)
    wait_out(NB - 1, 1)


@jax.jit
def _permute(z, perm_i32):
    k = functools.partial(
        pl.kernel,
        out_type=jax.ShapeDtypeStruct((N_ROWS, N_COLS), jnp.float32),
        mesh=plsc.VectorSubcoreMesh(core_axis_name="c", subcore_axis_name="s"),
        scratch_types=[
            pltpu.VMEM((N_COLS,), jnp.int32),
            pltpu.VMEM((R, N_COLS), jnp.float32),
            pltpu.VMEM((R, N_COLS), jnp.float32),
            pltpu.VMEM((R, N_COLS), jnp.float32),
            pltpu.VMEM((R, N_COLS), jnp.float32),
            pltpu.SemaphoreType.DMA,
            pltpu.SemaphoreType.DMA,
            pltpu.SemaphoreType.DMA,
            pltpu.SemaphoreType.DMA,
        ],
        compiler_params=pltpu.CompilerParams(
            use_tc_tiling_on_sc=False, needs_layout_passes=False
        ),
    )(_body)
    return k(z, perm_i32)


def kernel(z, perm):
    return _permute(z, perm.astype(jnp.int32))


# speedup vs baseline: 5.1065x; 2.7439x over previous
import functools

import jax
import jax.numpy as jnp
from jax import lax
from jax.experimental import pallas as pl
from jax.experimental.pallas import tpu as pltpu
from jax.experimental.pallas import tpu_sc as plsc

N_ROWS = 16384
N_COLS = 2048
NC = 2
NS = 16
NW = NC * NS
N_TROWS = N_ROWS // 8
TROWS_PER_W = N_TROWS // NW
TROW_WORDS = 8 * N_COLS
JV = N_COLS // 16


def _body(z_hbm, perm_hbm, out_hbm,
          perm_v, pcol_v, in_v0, in_v1, out_v0, out_v1, out_v2, out_v3,
          sin0, sin1, sout0, sout1, sout2, sout3):
    wid = lax.axis_index("s") * NC + lax.axis_index("c")
    base_t = wid * TROWS_PER_W

    pltpu.sync_copy(perm_hbm, perm_v)

    @plsc.parallel_loop(0, JV, 1, unroll=4)
    def _pcol(jv):
        p = perm_v[pl.ds(jv * 16, 16)]
        pcol_v[pl.ds(jv * 16, 16)] = ((p >> 7) << 10) | (p & 127)

    in_bufs = (in_v0, in_v1)
    out_bufs = (out_v0, out_v1, out_v2, out_v3)
    sins = (sin0, sin1)
    souts = (sout0, sout1, sout2, sout3)

    def start_in(b, d):
        pltpu.async_copy(z_hbm.at[base_t + b], in_bufs[d], sins[d])

    def wait_in(b, d):
        pltpu.make_async_copy(z_hbm.at[base_t + b], in_bufs[d], sins[d]).wait()

    def start_out(b, d):
        pltpu.async_copy(out_bufs[d], out_hbm.at[base_t + b], souts[d])

    def wait_out(b, d):
        pltpu.make_async_copy(
            out_bufs[d], out_hbm.at[base_t + b], souts[d]).wait()

    start_in(0, 0)

    def block_quad(b4, _):
        for k in range(4):
            b = 4 * b4 + k
            din = k % 2
            wait_in(b, din)

            @pl.when(b + 1 < TROWS_PER_W)
            def _():
                start_in(b + 1, 1 - din)

            @pl.when(b4 > 0)
            def _():
                wait_out(b - 4, k)

            ov = out_bufs[k]
            iv = in_bufs[din]

            @plsc.parallel_loop(0, JV, 1, unroll=8)
            def _jvec(jv):
                col0 = jv * 16
                idx = pcol_v[pl.ds(col0, 16)]
                obase = ((jv >> 3) << 10) | ((jv & 7) * 16)
                for il in range(8):
                    ov[pl.ds(obase + il * 128, 16)] = plsc.load_gather(
                        iv, [idx + il * 128])

            start_out(b, k)
        return 0

    lax.fori_loop(0, TROWS_PER_W // 4, block_quad, 0)
    for k in range(4):
        wait_out(TROWS_PER_W - 4 + k, k)


@jax.jit
def _permute(z_tiled, perm_i32):
    k = functools.partial(
        pl.kernel,
        out_type=jax.ShapeDtypeStruct((N_TROWS, TROW_WORDS), jnp.float32),
        mesh=plsc.VectorSubcoreMesh(core_axis_name="c", subcore_axis_name="s"),
        scratch_types=[
            pltpu.VMEM((N_COLS,), jnp.int32),
            pltpu.VMEM((N_COLS,), jnp.int32),
            pltpu.VMEM((TROW_WORDS,), jnp.float32),
            pltpu.VMEM((TROW_WORDS,), jnp.float32),
            pltpu.VMEM((TROW_WORDS,), jnp.float32),
            pltpu.VMEM((TROW_WORDS,), jnp.float32),
            pltpu.VMEM((TROW_WORDS,), jnp.float32),
            pltpu.VMEM((TROW_WORDS,), jnp.float32),
            pltpu.SemaphoreType.DMA,
            pltpu.SemaphoreType.DMA,
            pltpu.SemaphoreType.DMA,
            pltpu.SemaphoreType.DMA,
            pltpu.SemaphoreType.DMA,
            pltpu.SemaphoreType.DMA,
        ],
        compiler_params=pltpu.CompilerParams(
            use_tc_tiling_on_sc=False, needs_layout_passes=False
        ),
    )(_body)
    return k(z_tiled, perm_i32)


def kernel(z, perm):
    z_tiled = jnp.transpose(
        z.reshape(N_TROWS, 8, N_COLS // 128, 128), (0, 2, 1, 3)
    ).reshape(N_TROWS, TROW_WORDS)
    out_tiled = _permute(z_tiled, perm.astype(jnp.int32))
    return jnp.transpose(
        out_tiled.reshape(N_TROWS, N_COLS // 128, 8, 128), (0, 2, 1, 3)
    ).reshape(N_ROWS, N_COLS)


# speedup vs baseline: 5.1745x; 1.0133x over previous
import functools

import jax
import jax.numpy as jnp
from jax import lax
from jax.experimental import pallas as pl
from jax.experimental.pallas import tpu as pltpu
from jax.experimental.pallas import tpu_sc as plsc

N_ROWS = 16384
N_COLS = 2048
NC = 2
NS = 16
NW = NC * NS
N_TROWS = N_ROWS // 8
TROWS_PER_W = N_TROWS // NW
TROW_WORDS = 8 * N_COLS
JV = N_COLS // 16


def _body(z_hbm, perm_hbm, out_hbm,
          perm_v, pcol_v, in_v0, in_v1, out_v0, out_v1,
          sin0, sin1, sout0, sout1):
    wid = lax.axis_index("s") * NC + lax.axis_index("c")
    base_t = wid * TROWS_PER_W

    in_bufs = (in_v0, in_v1)
    out_bufs = (out_v0, out_v1)
    sins = (sin0, sin1)
    souts = (sout0, sout1)

    def start_in(b, d):
        pltpu.async_copy(z_hbm.at[base_t + b], in_bufs[d], sins[d])

    def wait_in(b, d):
        pltpu.make_async_copy(z_hbm.at[base_t + b], in_bufs[d], sins[d]).wait()

    def start_out(b, d):
        pltpu.async_copy(out_bufs[d], out_hbm.at[base_t + b], souts[d])

    def wait_out(b, d):
        pltpu.make_async_copy(
            out_bufs[d], out_hbm.at[base_t + b], souts[d]).wait()

    start_in(0, 0)
    pltpu.sync_copy(perm_hbm, perm_v)

    @plsc.parallel_loop(0, JV, 1, unroll=4)
    def _pcol(jv):
        p = perm_v[pl.ds(jv * 16, 16)]
        pcol_v[pl.ds(jv * 16, 16)] = ((p >> 7) << 10) | (p & 127)

    def block_pair(b2, _):
        for d in range(2):
            b = 2 * b2 + d
            wait_in(b, d)

            @pl.when(b + 1 < TROWS_PER_W)
            def _():
                start_in(b + 1, 1 - d)

            @pl.when(b2 > 0)
            def _():
                wait_out(b - 2, d)

            ov = out_bufs[d]
            iv = in_bufs[d]

            @plsc.parallel_loop(0, JV, 1, unroll=8)
            def _jvec(jv):
                col0 = jv * 16
                idx = pcol_v[pl.ds(col0, 16)]
                obase = ((jv >> 3) << 10) | ((jv & 7) * 16)
                for il in range(8):
                    ov[pl.ds(obase + il * 128, 16)] = plsc.load_gather(
                        iv, [idx + il * 128])

            start_out(b, d)
        return 0

    lax.fori_loop(0, TROWS_PER_W // 2, block_pair, 0)
    wait_out(TROWS_PER_W - 2, 0)
    wait_out(TROWS_PER_W - 1, 1)


@jax.jit
def _permute(z_tiled, perm_i32):
    k = functools.partial(
        pl.kernel,
        out_type=jax.ShapeDtypeStruct((N_TROWS, TROW_WORDS), jnp.float32),
        mesh=plsc.VectorSubcoreMesh(core_axis_name="c", subcore_axis_name="s"),
        scratch_types=[
            pltpu.VMEM((N_COLS,), jnp.int32),
            pltpu.VMEM((N_COLS,), jnp.int32),
            pltpu.VMEM((TROW_WORDS,), jnp.float32),
            pltpu.VMEM((TROW_WORDS,), jnp.float32),
            pltpu.VMEM((TROW_WORDS,), jnp.float32),
            pltpu.VMEM((TROW_WORDS,), jnp.float32),
            pltpu.SemaphoreType.DMA,
            pltpu.SemaphoreType.DMA,
            pltpu.SemaphoreType.DMA,
            pltpu.SemaphoreType.DMA,
        ],
        compiler_params=pltpu.CompilerParams(
            use_tc_tiling_on_sc=False, needs_layout_passes=False
        ),
    )(_body)
    return k(z_tiled, perm_i32)


def kernel(z, perm):
    z_tiled = jnp.transpose(
        z.reshape(N_TROWS, 8, N_COLS // 128, 128), (0, 2, 1, 3)
    ).reshape(N_TROWS, TROW_WORDS)
    out_tiled = _permute(z_tiled, perm.astype(jnp.int32))
    return jnp.transpose(
        out_tiled.reshape(N_TROWS, N_COLS // 128, 8, 128), (0, 2, 1, 3)
    ).reshape(N_ROWS, N_COLS)
